# Initial kernel scaffold; baseline (speedup 1.0000x reference)
import jax
import jax.numpy as jnp
from jax.experimental import pallas as pl


def kernel(z, elec_table, tables):
    raise NotImplementedError("write your pallas kernel here")


# speedup vs baseline: 7.1684x; 7.1684x over previous
import jax
import jax.numpy as jnp
from jax import lax
from jax.experimental import pallas as pl
from jax.experimental.pallas import tpu as pltpu
from jax.experimental.pallas import tpu_sc as plsc

_N_ORB = 19
_D = 128
_N_ELEM = 96
_MAX_E = 15
_N_ATOMS = 10000
_NC, _NS = 2, 16
_NW = _NC * _NS
_CHUNK = 16
_N_CHUNKS = _N_ATOMS // _CHUNK
_CHUNKS_PER_W = -(-_N_CHUNKS // _NW)
_EPS = _N_ELEM // _NS
_RPS = _EPS * _N_ORB
_RPAD = 120


def _sc_body(z_hbm, idx_hbm, tabs_hbm, out_hbm,
             e_sh, src_idx, rows_v, z_v, buf, sem):
    c = lax.axis_index("c")
    s = lax.axis_index("s")

    pltpu.sync_copy(idx_hbm.at[pl.ds(s * _RPAD, _RPAD)], src_idx)
    pltpu.sync_copy(tabs_hbm.at[src_idx], rows_v)
    for k in range(_EPS):
        pltpu.sync_copy(rows_v.at[pl.ds(k * _N_ORB, _N_ORB)],
                        e_sh.at[s * _EPS + k])
    plsc.subcore_barrier()

    wid = s * _NC + c
    for jj in range(_CHUNKS_PER_W):
        g = wid + _NW * jj

        @pl.when(g < _N_CHUNKS)
        def _():
            base = g * _CHUNK
            pltpu.sync_copy(z_hbm.at[pl.ds(base, _CHUNK)], z_v)
            pltpu.async_copy(e_sh.at[z_v], buf, sem).wait()
            pltpu.sync_copy(buf, out_hbm.at[pl.ds(base, _CHUNK)])


def kernel(z, elec_table, tables):
    pad_mask = jnp.ones((_MAX_E,), tables.dtype).at[0].set(0.0)
    tabs = (tables * pad_mask[None, :, None]).reshape(_N_ORB * _MAX_E, _D)
    idx = (elec_table.astype(jnp.int32)
           + (jnp.arange(_N_ORB, dtype=jnp.int32) * _MAX_E)[None, :])
    idx = jnp.pad(idx.reshape(_NS, _RPS), ((0, 0), (0, _RPAD - _RPS)))
    idx = idx.reshape(-1)
    z = z.astype(jnp.int32)

    mesh = plsc.VectorSubcoreMesh(core_axis_name="c", subcore_axis_name="s",
                                  num_cores=_NC, num_subcores=_NS)
    run = pl.kernel(
        _sc_body,
        out_type=jax.ShapeDtypeStruct((_N_ATOMS, _N_ORB, _D), jnp.float32),
        mesh=mesh,
        compiler_params=pltpu.CompilerParams(use_tc_tiling_on_sc=False),
        scratch_types=[
            pltpu.VMEM_SHARED((_N_ELEM, _N_ORB, _D), jnp.float32),
            pltpu.VMEM((_RPAD,), jnp.int32),
            pltpu.VMEM((_RPAD, _D), jnp.float32),
            pltpu.VMEM((_CHUNK,), jnp.int32),
            pltpu.VMEM((_CHUNK, _N_ORB, _D), jnp.float32),
            pltpu.SemaphoreType.DMA,
        ],
    )
    return run(z, idx, tabs)


# speedup vs baseline: 7.8588x; 1.0963x over previous
import jax
import jax.numpy as jnp
from jax import lax
from jax.experimental import pallas as pl
from jax.experimental.pallas import tpu as pltpu
from jax.experimental.pallas import tpu_sc as plsc

_N_ORB = 19
_D = 128
_N_ELEM = 96
_MAX_E = 15
_N_ATOMS = 10000
_NC, _NS = 2, 16
_NW = _NC * _NS
_CHUNK = 16
_N_CHUNKS = _N_ATOMS // _CHUNK
_CHUNKS_PER_W = -(-_N_CHUNKS // _NW)
_EPS = _N_ELEM // _NS
_RPS = _EPS * _N_ORB
_RPAD = 120


_APW = _CHUNKS_PER_W * _CHUNK


def _sc_body(z_hbm, idx_hbm, tabs_hbm, out_hbm,
             e_sh, src_idx, rows_v, z_all, buf0, buf1,
             sem, sz, sg0, sg1, sw0, sw1):
    c = lax.axis_index("c")
    s = lax.axis_index("s")
    wid = s * _NC + c

    zbase = jnp.minimum(wid * _APW, _N_ATOMS - _APW)
    zd = pltpu.async_copy(z_hbm.at[pl.ds(zbase, _APW)], z_all, sz)

    pltpu.sync_copy(idx_hbm.at[pl.ds(s * _RPAD, _RPAD)], src_idx)
    pltpu.sync_copy(tabs_hbm.at[src_idx], rows_v)
    for k in range(_EPS):
        pltpu.sync_copy(rows_v.at[pl.ds(k * _N_ORB, _N_ORB)],
                        e_sh.at[s * _EPS + k])
    plsc.subcore_barrier()
    zd.wait()

    bufs, sgs, sws = (buf0, buf1), (sg0, sg1), (sw0, sw1)

    def g_of(jj):
        return jnp.minimum(wid * _CHUNKS_PER_W + jj, _N_CHUNKS - 1)

    def start_gather(jj):
        idx = z_all.at[pl.ds(g_of(jj) * _CHUNK - zbase, _CHUNK)]
        return pltpu.async_copy(e_sh.at[idx], bufs[jj % 2], sgs[jj % 2])

    gd = [start_gather(0), None]
    wd = [None, None]
    for jj in range(_CHUNKS_PER_W):
        b = jj % 2
        gd[b].wait()
        wd[b] = pltpu.async_copy(
            bufs[b], out_hbm.at[pl.ds(g_of(jj) * _CHUNK, _CHUNK)], sws[b])
        if jj + 1 < _CHUNKS_PER_W:
            if wd[1 - b] is not None:
                wd[1 - b].wait()
            gd[1 - b] = start_gather(jj + 1)
    wd[0].wait()
    wd[1].wait()


def kernel(z, elec_table, tables):
    pad_mask = jnp.ones((_MAX_E,), tables.dtype).at[0].set(0.0)
    tabs = (tables * pad_mask[None, :, None]).reshape(_N_ORB * _MAX_E, _D)
    idx = (elec_table.astype(jnp.int32)
           + (jnp.arange(_N_ORB, dtype=jnp.int32) * _MAX_E)[None, :])
    idx = jnp.pad(idx.reshape(_NS, _RPS), ((0, 0), (0, _RPAD - _RPS)))
    idx = idx.reshape(-1)
    z = z.astype(jnp.int32)

    mesh = plsc.VectorSubcoreMesh(core_axis_name="c", subcore_axis_name="s",
                                  num_cores=_NC, num_subcores=_NS)
    run = pl.kernel(
        _sc_body,
        out_type=jax.ShapeDtypeStruct((_N_ATOMS, _N_ORB, _D), jnp.float32),
        mesh=mesh,
        compiler_params=pltpu.CompilerParams(use_tc_tiling_on_sc=False),
        scratch_types=[
            pltpu.VMEM_SHARED((_N_ELEM, _N_ORB, _D), jnp.float32),
            pltpu.VMEM((_RPAD,), jnp.int32),
            pltpu.VMEM((_RPAD, _D), jnp.float32),
            pltpu.VMEM((_APW,), jnp.int32),
            pltpu.VMEM((_CHUNK, _N_ORB, _D), jnp.float32),
            pltpu.VMEM((_CHUNK, _N_ORB, _D), jnp.float32),
            pltpu.SemaphoreType.DMA,
            pltpu.SemaphoreType.DMA,
            pltpu.SemaphoreType.DMA,
            pltpu.SemaphoreType.DMA,
            pltpu.SemaphoreType.DMA,
            pltpu.SemaphoreType.DMA,
        ],
    )
    return run(z, idx, tabs)


# speedup vs baseline: 13.7802x; 1.7535x over previous
import jax
import jax.numpy as jnp
from jax import lax
from jax.experimental import pallas as pl
from jax.experimental.pallas import tpu as pltpu
from jax.experimental.pallas import tpu_sc as plsc

_N_ORB = 19
_OP = 24
_D = 128
_N_ELEM = 96
_MAX_E = 15
_N_ATOMS = 10000
_NC, _NS = 2, 16
_NW = _NC * _NS
_CHUNK = 16
_N_CHUNKS = _N_ATOMS // _CHUNK
_CHUNKS_PER_W = -(-_N_CHUNKS // _NW)
_APW = _CHUNKS_PER_W * _CHUNK
_EPS = _N_ELEM // _NS
_IPS = _EPS * _OP


def _sc_body(z_hbm, idx_hbm, tabs_hbm, out_hbm,
             e_sh, src_idx, rows_v, z_all, buf0, buf1,
             sem, sz, sg0, sg1, sw0, sw1):
    c = lax.axis_index("c")
    s = lax.axis_index("s")
    wid = s * _NC + c

    zbase = jnp.minimum(wid * _APW, _N_ATOMS - _APW)
    zd = pltpu.async_copy(z_hbm.at[pl.ds(zbase, _APW)], z_all, sz)

    for h in range(2):
        pltpu.sync_copy(idx_hbm.at[pl.ds(s * _IPS + h * 72, 72)], src_idx)
        pltpu.async_copy(tabs_hbm.at[src_idx], rows_v, sem).wait()
        for k in range(3):
            pltpu.sync_copy(rows_v.at[pl.ds(k * _OP, _OP)],
                            e_sh.at[s * _EPS + h * 3 + k])
    plsc.subcore_barrier()
    zd.wait()

    bufs, sgs, sws = (buf0, buf1), (sg0, sg1), (sw0, sw1)

    def g_of(jj):
        return jnp.minimum(wid * _CHUNKS_PER_W + jj, _N_CHUNKS - 1)

    def start_gather(jj):
        idx = z_all.at[pl.ds(g_of(jj) * _CHUNK - zbase, _CHUNK)]
        return pltpu.async_copy(e_sh.at[idx], bufs[jj % 2], sgs[jj % 2])

    gd = [start_gather(0), None]
    wd = [[], []]
    for jj in range(_CHUNKS_PER_W):
        b = jj % 2
        gd[b].wait()
        base = g_of(jj) * _CHUNK
        wd[b] = [
            pltpu.async_copy(bufs[b].at[pl.ds(a, 1), pl.ds(0, _N_ORB)],
                             out_hbm.at[pl.ds(base + a, 1)], sws[b])
            for a in range(_CHUNK)
        ]
        if jj + 1 < _CHUNKS_PER_W:
            for d in wd[1 - b]:
                d.wait()
            wd[1 - b] = []
            gd[1 - b] = start_gather(jj + 1)
    for lst in wd:
        for d in lst:
            d.wait()


def kernel(z, elec_table, tables):
    pad_mask = jnp.ones((_MAX_E,), tables.dtype).at[0].set(0.0)
    tabs = (tables * pad_mask[None, :, None]).reshape(_N_ORB * _MAX_E, _D)
    idx = (elec_table.astype(jnp.int32)
           + (jnp.arange(_N_ORB, dtype=jnp.int32) * _MAX_E)[None, :])
    idx = jnp.pad(idx, ((0, 0), (0, _OP - _N_ORB))).reshape(-1)
    z = z.astype(jnp.int32)

    mesh = plsc.VectorSubcoreMesh(core_axis_name="c", subcore_axis_name="s",
                                  num_cores=_NC, num_subcores=_NS)
    run = pl.kernel(
        _sc_body,
        out_type=jax.ShapeDtypeStruct((_N_ATOMS, _N_ORB, _D), jnp.float32),
        mesh=mesh,
        scratch_types=[
            pltpu.VMEM_SHARED((_N_ELEM, _OP, _D), jnp.float32),
            pltpu.VMEM((72,), jnp.int32),
            pltpu.VMEM((72, _D), jnp.float32),
            pltpu.VMEM((_APW,), jnp.int32),
            pltpu.VMEM((_CHUNK, _OP, _D), jnp.float32),
            pltpu.VMEM((_CHUNK, _OP, _D), jnp.float32),
            pltpu.SemaphoreType.DMA,
            pltpu.SemaphoreType.DMA,
            pltpu.SemaphoreType.DMA,
            pltpu.SemaphoreType.DMA,
            pltpu.SemaphoreType.DMA,
            pltpu.SemaphoreType.DMA,
        ],
    )
    return run(z, idx, tabs)
